# baseline (speedup 1.0000x reference)
import functools

import jax
import jax.numpy as jnp
import numpy as np
from jax import lax
from jax.experimental import pallas as pl
from jax.experimental.pallas import tpu as pltpu
from jax.experimental.pallas import tpu_sc as plsc

_EMBED = 256
_HALF = 128
_KS = 1024
_KC = 512
_N = 18432
_BLK = 1024
_GRID = _N // _BLK

_NC = 2
_NS = 16
_NW = _NC * _NS
_BPW = _N // _NW


def _argmin_lanes(d, k, lane_iota):
    nch = k // 128
    val = d[:, 0:128]
    cb = jnp.zeros(val.shape, jnp.int32)
    for c in range(1, nch):
        ch = d[:, c * 128:(c + 1) * 128]
        cond = ch < val
        val = jnp.where(cond, ch, val)
        cb = jnp.where(cond, jnp.int32(c), cb)
    val_t = val.T
    cb_t = cb.T
    v_t = jnp.min(val_t, axis=0, keepdims=True)
    jl_t = cb_t * jnp.int32(128) + lane_iota
    idx = jnp.min(jnp.where(val_t == v_t, jl_t, jnp.int32(2**30)), axis=0)
    return v_t, idx


def _row_norm(x):
    t = (x * x).T
    w = _HALF
    while w > 1:
        w //= 2
        t = t[:w, :] + t[w:2 * w, :]
    return t.T


def _dist_body(x_ref, w2s_ref, w2c_ref, ws_ref, wc_ref,
               w2s_col_ref, w2c_col_ref, lane_ref,
               sidx_ref, cidx_ref, loss_ref, rep_ref):
    xs = x_ref[:, :_HALF]
    xc = x_ref[:, _HALF:]
    lane = lane_ref[...]
    x2s = _row_norm(xs)
    x2c = _row_norm(xc)
    wsn = ws_ref[...] * jnp.float32(-2.0)
    wcn = wc_ref[...] * jnp.float32(-2.0)

    mms = lax.dot_general(xs, wsn, (((1,), (1,)), ((), ())),
                          preferred_element_type=jnp.float32)
    ds = (x2s + w2s_ref[...]) + mms
    vs, sidx = _argmin_lanes(ds, _KS, lane)
    sidx_ref[0, 0, :] = sidx

    mmc = lax.dot_general(xc, wcn, (((1,), (1,)), ((), ())),
                          preferred_element_type=jnp.float32)
    dc = (x2c + w2c_ref[...]) + mmc
    vc, cidx = _argmin_lanes(dc, _KC, lane)
    cidx_ref[0, 0, :] = cidx

    @pl.when(pl.program_id(0) == 0)
    def _():
        loss_ref[...] = jnp.zeros_like(loss_ref)
        rep_s = _rep_half(ws_ref[...], w2s_col_ref[...], w2s_ref[...], _KS)
        rep_c = _rep_half(wc_ref[...], w2c_col_ref[...], w2c_ref[...], _KC)
        rep_ref[...] = jnp.reshape((rep_s + rep_c) * jnp.float32(0.1), (1, 1))

    tot = jnp.sum(vs) + jnp.sum(vc)
    loss_ref[...] += jnp.reshape(tot, (1, 1))

    @pl.when(pl.program_id(0) == _GRID - 1)
    def _():
        loss_ref[...] = loss_ref[...] * jnp.float32(1.25 / (_N * _EMBED))


def _rep_half(w, w2col, w2row, k):
    g = lax.dot_general(w, w, (((1,), (1,)), ((), ())),
                        preferred_element_type=jnp.float32)
    d = (w2col + w2row) - 2.0 * g
    r = lax.broadcasted_iota(jnp.int32, d.shape, 0)
    c = lax.broadcasted_iota(jnp.int32, d.shape, 1)
    d = d + jnp.where(r == c, jnp.float32(1e-05), jnp.float32(0.0))
    rep = jnp.sum(1.0 / (d + 0.0001)) - jnp.float32(k * (1.0 / 0.0001))
    return rep / jnp.float32(k * (k - 1))


@functools.partial(
    pl.kernel,
    out_type=[
        jax.ShapeDtypeStruct((_N, _EMBED), jnp.float32),
        jax.ShapeDtypeStruct((_N // _BPW, _BPW), jnp.int32),
        jax.ShapeDtypeStruct((_N // _BPW, _BPW), jnp.int32),
    ],
    mesh=plsc.VectorSubcoreMesh(core_axis_name="c", subcore_axis_name="s"),
    scratch_types=[
        pltpu.VMEM((_BPW,), jnp.int32),
        pltpu.VMEM((_BPW, _HALF), jnp.float32),
        pltpu.SemaphoreType.DMA,
    ],
)
def _sc_gather(ws_hbm, wc_hbm, sidx_hbm, cidx_hbm,
               out_hbm, sidx_out, cidx_out, idx_v, rows_v, sem):
    wid = lax.axis_index("s") * _NC + lax.axis_index("c")
    base = wid * _BPW
    pltpu.sync_copy(sidx_hbm.at[pl.ds(base, _BPW)], idx_v)
    pltpu.async_copy(ws_hbm.at[idx_v], rows_v, sem).wait()
    pltpu.sync_copy(rows_v, out_hbm.at[pl.ds(base, _BPW), pl.ds(0, _HALF)])
    pltpu.sync_copy(idx_v, sidx_out.at[wid])
    pltpu.sync_copy(cidx_hbm.at[pl.ds(base, _BPW)], idx_v)
    pltpu.async_copy(wc_hbm.at[idx_v], rows_v, sem).wait()
    pltpu.sync_copy(rows_v, out_hbm.at[pl.ds(base, _BPW), pl.ds(_HALF, _HALF)])
    pltpu.sync_copy(idx_v, cidx_out.at[wid])


def kernel(inputs, W_shape, W_color):
    b, k, d = inputs.shape
    flat = inputs.reshape(-1, _EMBED)

    w2s = jnp.sum(W_shape ** 2, axis=1)
    w2c = jnp.sum(W_color ** 2, axis=1)

    sidx3, cidx3, loss_sum, rep_out = pl.pallas_call(
        _dist_body,
        grid=(_GRID,),
        in_specs=[
            pl.BlockSpec((_BLK, _EMBED), lambda i: (i, 0)),
            pl.BlockSpec((1, _KS), lambda i: (0, 0)),
            pl.BlockSpec((1, _KC), lambda i: (0, 0)),
            pl.BlockSpec((_KS, _HALF), lambda i: (0, 0)),
            pl.BlockSpec((_KC, _HALF), lambda i: (0, 0)),
            pl.BlockSpec((_KS, 1), lambda i: (0, 0)),
            pl.BlockSpec((_KC, 1), lambda i: (0, 0)),
            pl.BlockSpec((128, 1), lambda i: (0, 0)),
        ],
        out_specs=[
            pl.BlockSpec((1, 1, _BLK), lambda i: (i, 0, 0)),
            pl.BlockSpec((1, 1, _BLK), lambda i: (i, 0, 0)),
            pl.BlockSpec((1, 1), lambda i: (0, 0)),
            pl.BlockSpec((1, 1), lambda i: (0, 0)),
        ],
        out_shape=[
            jax.ShapeDtypeStruct((_GRID, 1, _BLK), jnp.int32),
            jax.ShapeDtypeStruct((_GRID, 1, _BLK), jnp.int32),
            jax.ShapeDtypeStruct((1, 1), jnp.float32),
            jax.ShapeDtypeStruct((1, 1), jnp.float32),
        ],
    )(flat, w2s.reshape(1, _KS), w2c.reshape(1, _KC),
      W_shape, W_color, w2s.reshape(_KS, 1), w2c.reshape(_KC, 1),
      np.arange(128, dtype=np.int32).reshape(128, 1))

    s_idx = sidx3.reshape(-1)
    c_idx = cidx3.reshape(-1)

    quantized_flat, sidx_bk, cidx_bk = _sc_gather(W_shape, W_color,
                                                  s_idx, c_idx)

    vq_loss = loss_sum[0, 0]
    rep_loss = rep_out[0, 0]
    quantized = quantized_flat.reshape(b, k, d)
    return (quantized, vq_loss, rep_loss, sidx_bk, cidx_bk)


# speedup vs baseline: 1.0565x; 1.0565x over previous
import functools

import jax
import jax.numpy as jnp
import numpy as np
from jax import lax
from jax.experimental import pallas as pl
from jax.experimental.pallas import tpu as pltpu
from jax.experimental.pallas import tpu_sc as plsc

_EMBED = 256
_HALF = 128
_KS = 1024
_KC = 512
_N = 18432
_BLK = 3072
_GRID = _N // _BLK

_NC = 2
_NS = 16
_NW = _NC * _NS
_BPW = _N // _NW


def _argmin_lanes(d, k, lane_iota):
    nch = k // 128
    val = d[:, 0:128]
    cb = jnp.zeros(val.shape, jnp.int32)
    for c in range(1, nch):
        ch = d[:, c * 128:(c + 1) * 128]
        cond = ch < val
        val = jnp.where(cond, ch, val)
        cb = jnp.where(cond, jnp.int32(c), cb)
    val_t = val.T
    cb_t = cb.T
    v_t = jnp.min(val_t, axis=0, keepdims=True)
    jl_t = cb_t * jnp.int32(128) + lane_iota
    idx = jnp.min(jnp.where(val_t == v_t, jl_t, jnp.int32(2**30)), axis=0)
    return v_t, idx


def _row_norm(x):
    t = (x * x).T
    w = _HALF
    while w > 1:
        w //= 2
        t = t[:w, :] + t[w:2 * w, :]
    return t.T


def _dist_body(x_ref, w2s_ref, w2c_ref, ws_ref, wc_ref, lane_ref,
               sidx_ref, cidx_ref, loss_ref, rep_ref):
    xs = x_ref[:, :_HALF]
    xc = x_ref[:, _HALF:]
    lane = lane_ref[...]
    x2s = _row_norm(xs)
    x2c = _row_norm(xc)
    wsn = ws_ref[...] * jnp.float32(-2.0)
    wcn = wc_ref[...] * jnp.float32(-2.0)

    mms = lax.dot_general(xs, wsn, (((1,), (1,)), ((), ())),
                          preferred_element_type=jnp.float32)
    ds = (x2s + w2s_ref[...]) + mms
    vs, sidx = _argmin_lanes(ds, _KS, lane)
    sidx_ref[0, 0, :] = sidx

    mmc = lax.dot_general(xc, wcn, (((1,), (1,)), ((), ())),
                          preferred_element_type=jnp.float32)
    dc = (x2c + w2c_ref[...]) + mmc
    vc, cidx = _argmin_lanes(dc, _KC, lane)
    cidx_ref[0, 0, :] = cidx

    @pl.when(pl.program_id(0) == 0)
    def _():
        loss_ref[...] = jnp.zeros_like(loss_ref)
        rep_s = _rep_half(ws_ref[...], w2s_ref[...].T, w2s_ref[...], _KS)
        rep_c = _rep_half(wc_ref[...], w2c_ref[...].T, w2c_ref[...], _KC)
        rep_ref[...] = jnp.reshape((rep_s + rep_c) * jnp.float32(0.1), (1, 1))

    tot = jnp.sum(vs) + jnp.sum(vc)
    loss_ref[...] += jnp.reshape(tot, (1, 1))

    @pl.when(pl.program_id(0) == _GRID - 1)
    def _():
        loss_ref[...] = loss_ref[...] * jnp.float32(1.25 / (_N * _EMBED))


def _rep_half(w, w2col, w2row, k):
    g = lax.dot_general(w, w, (((1,), (1,)), ((), ())),
                        preferred_element_type=jnp.float32)
    d = (w2col + w2row) - 2.0 * g
    r = lax.broadcasted_iota(jnp.int32, d.shape, 0)
    c = lax.broadcasted_iota(jnp.int32, d.shape, 1)
    d = d + jnp.where(r == c, jnp.float32(1e-05), jnp.float32(0.0))
    rep = jnp.sum(1.0 / (d + 0.0001)) - jnp.float32(k * (1.0 / 0.0001))
    return rep / jnp.float32(k * (k - 1))


@functools.partial(
    pl.kernel,
    out_type=[
        jax.ShapeDtypeStruct((_N, _EMBED), jnp.float32),
        jax.ShapeDtypeStruct((_N // _BPW, _BPW), jnp.int32),
        jax.ShapeDtypeStruct((_N // _BPW, _BPW), jnp.int32),
    ],
    mesh=plsc.VectorSubcoreMesh(core_axis_name="c", subcore_axis_name="s"),
    scratch_types=[
        pltpu.VMEM((_BPW,), jnp.int32),
        pltpu.VMEM((_BPW, _HALF), jnp.float32),
        pltpu.SemaphoreType.DMA,
    ],
)
def _sc_gather(ws_hbm, wc_hbm, sidx_hbm, cidx_hbm,
               out_hbm, sidx_out, cidx_out, idx_v, rows_v, sem):
    wid = lax.axis_index("s") * _NC + lax.axis_index("c")
    base = wid * _BPW
    pltpu.sync_copy(sidx_hbm.at[pl.ds(base, _BPW)], idx_v)
    pltpu.async_copy(ws_hbm.at[idx_v], rows_v, sem).wait()
    pltpu.sync_copy(rows_v, out_hbm.at[pl.ds(base, _BPW), pl.ds(0, _HALF)])
    pltpu.sync_copy(idx_v, sidx_out.at[wid])
    pltpu.sync_copy(cidx_hbm.at[pl.ds(base, _BPW)], idx_v)
    pltpu.async_copy(wc_hbm.at[idx_v], rows_v, sem).wait()
    pltpu.sync_copy(rows_v, out_hbm.at[pl.ds(base, _BPW), pl.ds(_HALF, _HALF)])
    pltpu.sync_copy(idx_v, cidx_out.at[wid])


def kernel(inputs, W_shape, W_color):
    b, k, d = inputs.shape
    flat = inputs.reshape(-1, _EMBED)

    w2s = jnp.sum(W_shape ** 2, axis=1)
    w2c = jnp.sum(W_color ** 2, axis=1)

    sidx3, cidx3, loss_sum, rep_out = pl.pallas_call(
        _dist_body,
        grid=(_GRID,),
        in_specs=[
            pl.BlockSpec((_BLK, _EMBED), lambda i: (i, 0)),
            pl.BlockSpec((1, _KS), lambda i: (0, 0)),
            pl.BlockSpec((1, _KC), lambda i: (0, 0)),
            pl.BlockSpec((_KS, _HALF), lambda i: (0, 0)),
            pl.BlockSpec((_KC, _HALF), lambda i: (0, 0)),
            pl.BlockSpec((128, 1), lambda i: (0, 0)),
        ],
        out_specs=[
            pl.BlockSpec((1, 1, _BLK), lambda i: (i, 0, 0)),
            pl.BlockSpec((1, 1, _BLK), lambda i: (i, 0, 0)),
            pl.BlockSpec((1, 1), lambda i: (0, 0)),
            pl.BlockSpec((1, 1), lambda i: (0, 0)),
        ],
        out_shape=[
            jax.ShapeDtypeStruct((_GRID, 1, _BLK), jnp.int32),
            jax.ShapeDtypeStruct((_GRID, 1, _BLK), jnp.int32),
            jax.ShapeDtypeStruct((1, 1), jnp.float32),
            jax.ShapeDtypeStruct((1, 1), jnp.float32),
        ],
    )(flat, w2s.reshape(1, _KS), w2c.reshape(1, _KC),
      W_shape, W_color,
      np.arange(128, dtype=np.int32).reshape(128, 1))

    s_idx = sidx3.reshape(-1)
    c_idx = cidx3.reshape(-1)

    quantized_flat, sidx_bk, cidx_bk = _sc_gather(W_shape, W_color,
                                                  s_idx, c_idx)

    vq_loss = loss_sum[0, 0]
    rep_loss = rep_out[0, 0]
    quantized = quantized_flat.reshape(b, k, d)
    return (quantized, vq_loss, rep_loss, sidx_bk, cidx_bk)
